# baseline (speedup 1.0000x reference)
import functools

import jax
import jax.numpy as jnp
from jax import lax
from jax.experimental import pallas as pl
from jax.experimental.pallas import tpu as pltpu
from jax.experimental.pallas import tpu_sc as plsc

_N = 10000
_E = 320000
_D = 128
_NC = 2
_NS = 16
_NW = _NC * _NS
_ROWS = 79
_EPW = _ROWS * 128
_EPAD = _NW * _EPW - _E
_C = 16
_NCH = _EPW // _C
_HC = 32
_TCH = _NW * _EPW // _HC
_K0 = 316
_K1 = (_TCH - 16 * _K0) // 16
_KMAX = max(_K0, _K1)
_NP = 10112
_RPT = _NP // _NS

_mesh = plsc.VectorSubcoreMesh(core_axis_name="c", subcore_axis_name="s")

_RCHUNKS = [(o, min(16, _RPT - o)) for o in range(0, _RPT, 16)]
_RCHUNKS32 = [(o, min(_HC, _RPT - o)) for o in range(0, _RPT, _HC)]


def _idx_at(ref, j):
    return ref[j // 8, pl.ds((j % 8) * _C, _C)]


@functools.partial(
    pl.kernel,
    out_type=jax.ShapeDtypeStruct((_NC, _NP, 16), jnp.float32),
    mesh=_mesh,
    scratch_types=[
        pltpu.VMEM((8, 128), jnp.int32),
        pltpu.VMEM((_C, 16), jnp.float32),
        pltpu.VMEM_SHARED((_NP, 16), jnp.float32),
        pltpu.SemaphoreType.DMA,
        pltpu.SemaphoreType.DMA,
    ],
    compiler_params=pltpu.CompilerParams(use_tc_tiling_on_sc=False),
)
def _degree_kernel(dst_hbm, out_hbm, dst_v, buf, acc, sem0, sem1):
    c = lax.axis_index("c")
    s = lax.axis_index("s")
    w = c * _NS + s
    row0 = s * _RPT

    @pl.loop(0, _C)
    def _(i):
        buf[i, :] = jnp.zeros((16,), jnp.float32)

    for off, rows in _RCHUNKS:
        pltpu.sync_copy(buf.at[pl.ds(0, rows)], acc.at[pl.ds(row0 + off, rows)])

    @pl.loop(0, _C)
    def _(i):
        buf[i, :] = jnp.full((16,), 1.0, jnp.float32)

    plsc.subcore_barrier()

    def fire(j, sem):
        pltpu.async_copy(buf, acc.at[_idx_at(dst_v, j)], sem, add=True)

    def drain(j, sem):
        pltpu.make_async_copy(buf, acc.at[_idx_at(dst_v, j)], sem).wait()

    def block(nrows):
        nch = nrows * 8

        def body(_):
            fire(0, sem0)

            @pl.loop(0, (nch - 2) // 2)
            def _(i):
                j = i * 2
                fire(j + 1, sem1)
                drain(j, sem0)
                fire(j + 2, sem0)
                drain(j + 1, sem1)

            fire(nch - 1, sem1)
            drain(nch - 2, sem0)
            drain(nch - 1, sem1)

        return body

    @pl.loop(0, _ROWS // 8)
    def _(b):
        pltpu.sync_copy(dst_hbm.at[w, pl.ds(b * 8, 8)], dst_v)
        block(8)(None)

    pltpu.sync_copy(dst_hbm.at[w, pl.ds(_ROWS - _ROWS % 8, _ROWS % 8)],
                    dst_v.at[pl.ds(0, _ROWS % 8)])
    block(_ROWS % 8)(None)

    plsc.subcore_barrier()
    for off, rows in _RCHUNKS:
        pltpu.sync_copy(acc.at[pl.ds(row0 + off, rows)], buf.at[pl.ds(0, rows)])
        pltpu.sync_copy(buf.at[pl.ds(0, rows)],
                        out_hbm.at[c, pl.ds(row0 + off, rows)])


@functools.partial(
    pl.kernel,
    out_type=jax.ShapeDtypeStruct((_NC, _NP, _D), jnp.float32),
    mesh=_mesh,
    scratch_types=[
        pltpu.VMEM((_KMAX, _HC), jnp.int32),
        pltpu.VMEM((_KMAX, _HC), jnp.int32),
        pltpu.VMEM((_HC, _D), jnp.float32),
        pltpu.VMEM((_HC, _D), jnp.float32),
        pltpu.VMEM_SHARED((_NP, _D), jnp.float32),
        pltpu.SemaphoreType.DMA,
        pltpu.SemaphoreType.DMA,
    ],
    compiler_params=pltpu.CompilerParams(use_tc_tiling_on_sc=False),
)
def _hop_kernel(g_hbm, src_hbm, dst_hbm, out_hbm,
                src_v, dst_v, buf0, buf1, acc, sem0, sem1):
    c = lax.axis_index("c")
    s = lax.axis_index("s")
    row0 = s * _RPT

    @pl.loop(0, _HC)
    def _(i):
        for k in range(_D // 16):
            buf0[i, pl.ds(16 * k, 16)] = jnp.zeros((16,), jnp.float32)

    for off, rows in _RCHUNKS32:
        pltpu.sync_copy(buf0.at[pl.ds(0, rows)], acc.at[pl.ds(row0 + off, rows)])
    plsc.subcore_barrier()

    def gather(j, buf, sem):
        pltpu.async_copy(g_hbm.at[src_v.at[j]], buf, sem)

    def gwait(j, buf, sem):
        pltpu.make_async_copy(g_hbm.at[src_v.at[j]], buf, sem).wait()

    def scat(j, buf):
        pltpu.sync_copy(buf, acc.at[dst_v.at[j]], add=True)

    def pipeline(K, start):
        pltpu.sync_copy(src_hbm.at[pl.ds(start, K)], src_v.at[pl.ds(0, K)])
        pltpu.sync_copy(dst_hbm.at[pl.ds(start, K)], dst_v.at[pl.ds(0, K)])
        gather(0, buf0, sem0)

        @pl.loop(0, (K - 2) // 2)
        def _(i):
            j = i * 2
            gather(j + 1, buf1, sem1)
            gwait(j, buf0, sem0)
            scat(j, buf0)
            gather(j + 2, buf0, sem0)
            gwait(j + 1, buf1, sem1)
            scat(j + 1, buf1)

        gather(K - 1, buf1, sem1)
        gwait(K - 2, buf0, sem0)
        scat(K - 2, buf0)
        gwait(K - 1, buf1, sem1)
        scat(K - 1, buf1)

    @pl.when(c == 0)
    def _():
        pipeline(_K0, s * _K0)

    @pl.when(c == 1)
    def _():
        pipeline(_K1, 16 * _K0 + s * _K1)

    plsc.subcore_barrier()
    for off, rows in _RCHUNKS32:
        pltpu.sync_copy(acc.at[pl.ds(row0 + off, rows)],
                        buf0.at[pl.ds(0, rows)])
        pltpu.sync_copy(buf0.at[pl.ds(0, rows)],
                        out_hbm.at[c, pl.ds(row0 + off, rows)])


def _norm_block(d0, d1):
    deg = jnp.maximum(d0[:, 0:1] + d1[:, 0:1], 1.0)
    return lax.rsqrt(deg)


_BR = 400
_GRID = _N // _BR

_feat_spec = pl.BlockSpec((_BR, _D), lambda i: (i, 0))
_deg0_spec = pl.BlockSpec((1, _BR, 16), lambda i: (0, i, 0))
_deg1_spec = pl.BlockSpec((1, _BR, 16), lambda i: (1, i, 0))
_part0_spec = pl.BlockSpec((1, _BR, _D), lambda i: (0, i, 0))
_part1_spec = pl.BlockSpec((1, _BR, _D), lambda i: (1, i, 0))


def _prep_body(dp_ref0, dp_ref1, h_ref, g0_ref, s0_ref):
    nrm = _norm_block(dp_ref0[0], dp_ref1[0])
    h = h_ref[...]
    g0_ref[...] = h * nrm
    s0_ref[...] = h * (1.0 / 3.0)


_prep = pl.pallas_call(
    _prep_body,
    grid=(_GRID,),
    in_specs=[_deg0_spec, _deg1_spec, _feat_spec],
    out_specs=[_feat_spec, _feat_spec],
    out_shape=[
        jax.ShapeDtypeStruct((_N, _D), jnp.float32),
        jax.ShapeDtypeStruct((_N, _D), jnp.float32),
    ],
)


def _accum_body(p_ref0, p_ref1, dp_ref0, dp_ref1, s_ref, g_out_ref, s_out_ref):
    nrm = _norm_block(dp_ref0[0], dp_ref1[0])
    h = (p_ref0[0] + p_ref1[0]) * nrm
    g_out_ref[...] = h * nrm
    s_out_ref[...] = s_ref[...] + h * (1.0 / 3.0)


_accum = pl.pallas_call(
    _accum_body,
    grid=(_GRID,),
    in_specs=[_part0_spec, _part1_spec, _deg0_spec, _deg1_spec, _feat_spec],
    out_specs=[_feat_spec, _feat_spec],
    out_shape=[
        jax.ShapeDtypeStruct((_N, _D), jnp.float32),
        jax.ShapeDtypeStruct((_N, _D), jnp.float32),
    ],
)


@jax.jit
def kernel(features, edge_index):
    srcf = jnp.concatenate([edge_index[0], jnp.zeros((_EPAD,), jnp.int32)])
    dstf = jnp.concatenate([edge_index[1], jnp.full((_EPAD,), _N, jnp.int32)])
    src = srcf.reshape(_TCH, _HC)
    dst = dstf.reshape(_TCH, _HC)

    degp = _degree_kernel(dstf.reshape(_NW, _ROWS, 128))
    g, s = _prep(degp, degp, features)

    def _body(_, carry):
        g_c, s_c = carry
        p = _hop_kernel(g_c, src, dst)
        return _accum(p, p, degp, degp, s_c)

    _, s = lax.fori_loop(0, 2, _body, (g, s))
    return s


# speedup vs baseline: 1.1260x; 1.1260x over previous
import functools

import jax
import jax.numpy as jnp
from jax import lax
from jax.experimental import pallas as pl
from jax.experimental.pallas import tpu as pltpu
from jax.experimental.pallas import tpu_sc as plsc

_N = 10000
_E = 320000
_D = 128
_NC = 2
_NS = 16
_NW = _NC * _NS
_ROWS = 79
_EPW = _ROWS * 128
_EPAD = _NW * _EPW - _E
_C = 16
_NCH = _EPW // _C
_HC = 32
_TCH = _NW * _EPW // _HC
_K0 = 416
_K1 = (_TCH - 16 * _K0) // 16
_KMAX = max(_K0, _K1)
_NP = 10112
_RPT = _NP // _NS

_mesh = plsc.VectorSubcoreMesh(core_axis_name="c", subcore_axis_name="s")

_RCHUNKS = [(o, min(16, _RPT - o)) for o in range(0, _RPT, 16)]
_RCHUNKS32 = [(o, min(_HC, _RPT - o)) for o in range(0, _RPT, _HC)]


def _idx_at(ref, j):
    return ref[j // 8, pl.ds((j % 8) * _C, _C)]


@functools.partial(
    pl.kernel,
    out_type=jax.ShapeDtypeStruct((_NC, _NP, 16), jnp.float32),
    mesh=_mesh,
    scratch_types=[
        pltpu.VMEM((8, 128), jnp.int32),
        pltpu.VMEM((_C, 16), jnp.float32),
        pltpu.VMEM_SHARED((_NP, 16), jnp.float32),
        pltpu.SemaphoreType.DMA,
        pltpu.SemaphoreType.DMA,
    ],
    compiler_params=pltpu.CompilerParams(use_tc_tiling_on_sc=False),
)
def _degree_kernel(dst_hbm, out_hbm, dst_v, buf, acc, sem0, sem1):
    c = lax.axis_index("c")
    s = lax.axis_index("s")
    w = c * _NS + s
    row0 = s * _RPT

    @pl.loop(0, _C)
    def _(i):
        buf[i, :] = jnp.zeros((16,), jnp.float32)

    for off, rows in _RCHUNKS:
        pltpu.sync_copy(buf.at[pl.ds(0, rows)], acc.at[pl.ds(row0 + off, rows)])

    @pl.loop(0, _C)
    def _(i):
        buf[i, :] = jnp.full((16,), 1.0, jnp.float32)

    plsc.subcore_barrier()

    def fire(j, sem):
        pltpu.async_copy(buf, acc.at[_idx_at(dst_v, j)], sem, add=True)

    def drain(j, sem):
        pltpu.make_async_copy(buf, acc.at[_idx_at(dst_v, j)], sem).wait()

    def block(nrows):
        nch = nrows * 8

        def body(_):
            fire(0, sem0)

            @pl.loop(0, (nch - 2) // 2)
            def _(i):
                j = i * 2
                fire(j + 1, sem1)
                drain(j, sem0)
                fire(j + 2, sem0)
                drain(j + 1, sem1)

            fire(nch - 1, sem1)
            drain(nch - 2, sem0)
            drain(nch - 1, sem1)

        return body

    @pl.loop(0, _ROWS // 8)
    def _(b):
        pltpu.sync_copy(dst_hbm.at[w, pl.ds(b * 8, 8)], dst_v)
        block(8)(None)

    pltpu.sync_copy(dst_hbm.at[w, pl.ds(_ROWS - _ROWS % 8, _ROWS % 8)],
                    dst_v.at[pl.ds(0, _ROWS % 8)])
    block(_ROWS % 8)(None)

    plsc.subcore_barrier()
    for off, rows in _RCHUNKS:
        pltpu.sync_copy(acc.at[pl.ds(row0 + off, rows)], buf.at[pl.ds(0, rows)])
        pltpu.sync_copy(buf.at[pl.ds(0, rows)],
                        out_hbm.at[c, pl.ds(row0 + off, rows)])


@functools.partial(
    pl.kernel,
    out_type=jax.ShapeDtypeStruct((_NC, _NP, _D), jnp.float32),
    mesh=_mesh,
    scratch_types=[
        pltpu.VMEM((_KMAX, _HC), jnp.int32),
        pltpu.VMEM((_KMAX, _HC), jnp.int32),
        pltpu.VMEM((_HC, _D), jnp.float32),
        pltpu.VMEM((_HC, _D), jnp.float32),
        pltpu.VMEM_SHARED((_NP, _D), jnp.float32),
        pltpu.SemaphoreType.DMA,
        pltpu.SemaphoreType.DMA,
    ],
    compiler_params=pltpu.CompilerParams(use_tc_tiling_on_sc=False),
)
def _hop_kernel(g_hbm, src_hbm, dst_hbm, out_hbm,
                src_v, dst_v, buf0, buf1, acc, sem0, sem1):
    c = lax.axis_index("c")
    s = lax.axis_index("s")
    row0 = s * _RPT

    @pl.loop(0, _HC)
    def _(i):
        for k in range(_D // 16):
            buf0[i, pl.ds(16 * k, 16)] = jnp.zeros((16,), jnp.float32)

    for off, rows in _RCHUNKS32:
        pltpu.sync_copy(buf0.at[pl.ds(0, rows)], acc.at[pl.ds(row0 + off, rows)])
    plsc.subcore_barrier()

    def gather(j, buf, sem):
        pltpu.async_copy(g_hbm.at[src_v.at[j]], buf, sem)

    def gwait(j, buf, sem):
        pltpu.make_async_copy(g_hbm.at[src_v.at[j]], buf, sem).wait()

    def scat(j, buf):
        pltpu.sync_copy(buf, acc.at[dst_v.at[j]], add=True)

    def pipeline(K, start):
        pltpu.sync_copy(src_hbm.at[pl.ds(start, K)], src_v.at[pl.ds(0, K)])
        pltpu.sync_copy(dst_hbm.at[pl.ds(start, K)], dst_v.at[pl.ds(0, K)])
        gather(0, buf0, sem0)

        @pl.loop(0, (K - 2) // 2)
        def _(i):
            j = i * 2
            gather(j + 1, buf1, sem1)
            gwait(j, buf0, sem0)
            scat(j, buf0)
            gather(j + 2, buf0, sem0)
            gwait(j + 1, buf1, sem1)
            scat(j + 1, buf1)

        gather(K - 1, buf1, sem1)
        gwait(K - 2, buf0, sem0)
        scat(K - 2, buf0)
        gwait(K - 1, buf1, sem1)
        scat(K - 1, buf1)

    @pl.when(c == 0)
    def _():
        pipeline(_K0, s * _K0)

    @pl.when(c == 1)
    def _():
        pipeline(_K1, 16 * _K0 + s * _K1)

    plsc.subcore_barrier()
    for off, rows in _RCHUNKS32:
        pltpu.sync_copy(acc.at[pl.ds(row0 + off, rows)],
                        buf0.at[pl.ds(0, rows)])
        pltpu.sync_copy(buf0.at[pl.ds(0, rows)],
                        out_hbm.at[c, pl.ds(row0 + off, rows)])


def _norm_block(d0, d1):
    deg = jnp.maximum(d0[:, 0:1] + d1[:, 0:1], 1.0)
    return lax.rsqrt(deg)


_BR = 400
_GRID = _N // _BR

_feat_spec = pl.BlockSpec((_BR, _D), lambda i: (i, 0))
_deg0_spec = pl.BlockSpec((1, _BR, 16), lambda i: (0, i, 0))
_deg1_spec = pl.BlockSpec((1, _BR, 16), lambda i: (1, i, 0))
_part0_spec = pl.BlockSpec((1, _BR, _D), lambda i: (0, i, 0))
_part1_spec = pl.BlockSpec((1, _BR, _D), lambda i: (1, i, 0))


def _prep_body(dp_ref0, dp_ref1, h_ref, g0_ref, s0_ref):
    nrm = _norm_block(dp_ref0[0], dp_ref1[0])
    h = h_ref[...]
    g0_ref[...] = h * nrm
    s0_ref[...] = h * (1.0 / 3.0)


_prep = pl.pallas_call(
    _prep_body,
    grid=(_GRID,),
    in_specs=[_deg0_spec, _deg1_spec, _feat_spec],
    out_specs=[_feat_spec, _feat_spec],
    out_shape=[
        jax.ShapeDtypeStruct((_N, _D), jnp.float32),
        jax.ShapeDtypeStruct((_N, _D), jnp.float32),
    ],
)


def _accum_body(p_ref0, p_ref1, dp_ref0, dp_ref1, s_ref, g_out_ref, s_out_ref):
    nrm = _norm_block(dp_ref0[0], dp_ref1[0])
    h = (p_ref0[0] + p_ref1[0]) * nrm
    g_out_ref[...] = h * nrm
    s_out_ref[...] = s_ref[...] + h * (1.0 / 3.0)


_accum = pl.pallas_call(
    _accum_body,
    grid=(_GRID,),
    in_specs=[_part0_spec, _part1_spec, _deg0_spec, _deg1_spec, _feat_spec],
    out_specs=[_feat_spec, _feat_spec],
    out_shape=[
        jax.ShapeDtypeStruct((_N, _D), jnp.float32),
        jax.ShapeDtypeStruct((_N, _D), jnp.float32),
    ],
)


@jax.jit
def kernel(features, edge_index):
    srcf = jnp.concatenate([edge_index[0], jnp.zeros((_EPAD,), jnp.int32)])
    dstf = jnp.concatenate([edge_index[1], jnp.full((_EPAD,), _N, jnp.int32)])
    src = srcf.reshape(_TCH, _HC)
    dst = dstf.reshape(_TCH, _HC)

    degp = _degree_kernel(dstf.reshape(_NW, _ROWS, 128))
    g, s = _prep(degp, degp, features)

    def _body(_, carry):
        g_c, s_c = carry
        p = _hop_kernel(g_c, src, dst)
        return _accum(p, p, degp, degp, s_c)

    _, s = lax.fori_loop(0, 2, _body, (g, s))
    return s
